# baseline (device time: 67671 ns/iter reference)
import jax
import jax.numpy as jnp
from jax import lax
from jax.experimental import pallas as pl
from jax.experimental.pallas import tpu as pltpu


def kernel(O, Wo):
    B, S, H, D = O.shape
    K = H * D
    N = Wo.shape[1]
    S_half = S // 2

    O2 = O.reshape(B, S, K)

    def body(o_ref, w_ref, out_ref, send_buf, recv_buf, send_sem, recv_sem):
        my_x = lax.axis_index("x")
        my_y = lax.axis_index("y")
        my_z = lax.axis_index("z")
        peer = (1 - my_x, my_y, my_z)

        barrier = pltpu.get_barrier_semaphore()
        pl.semaphore_signal(
            barrier, inc=1, device_id=peer, device_id_type=pl.DeviceIdType.MESH
        )
        pl.semaphore_wait(barrier, 1)

        w = w_ref[...].astype(jnp.bfloat16)
        my_lo = my_x * S_half
        peer_lo = (1 - my_x) * S_half

        for b in range(B):
            a = o_ref[b, pl.ds(peer_lo, S_half)].astype(jnp.bfloat16)
            send_buf[b] = jnp.dot(
                a, w, preferred_element_type=jnp.float32
            ).astype(jnp.bfloat16)

        rdma = pltpu.make_async_remote_copy(
            src_ref=send_buf,
            dst_ref=recv_buf,
            send_sem=send_sem,
            recv_sem=recv_sem,
            device_id=peer,
            device_id_type=pl.DeviceIdType.MESH,
        )
        rdma.start()

        for b in range(B):
            a = o_ref[b, pl.ds(my_lo, S_half)].astype(jnp.bfloat16)
            out_ref[b] = jnp.dot(a, w, preferred_element_type=jnp.float32)

        rdma.wait()
        for b in range(B):
            out_ref[b] = out_ref[b] + recv_buf[b].astype(jnp.float32)

    return pl.pallas_call(
        body,
        out_shape=jax.ShapeDtypeStruct((B, S_half, N), jnp.float32),
        in_specs=[
            pl.BlockSpec(memory_space=pltpu.VMEM),
            pl.BlockSpec(memory_space=pltpu.VMEM),
        ],
        out_specs=pl.BlockSpec(memory_space=pltpu.VMEM),
        scratch_shapes=[
            pltpu.VMEM((B, S_half, N), jnp.bfloat16),
            pltpu.VMEM((B, S_half, N), jnp.bfloat16),
            pltpu.SemaphoreType.DMA,
            pltpu.SemaphoreType.DMA,
        ],
        compiler_params=pltpu.CompilerParams(collective_id=0),
    )(O2, Wo)


# device time: 63063 ns/iter; 1.0731x vs baseline; 1.0731x over previous
import jax
import jax.numpy as jnp
from jax import lax
from jax.experimental import pallas as pl
from jax.experimental.pallas import tpu as pltpu

_CHUNKS_PER_B = 4


def kernel(O, Wo):
    B, S, H, D = O.shape
    K = H * D
    N = Wo.shape[1]
    S_half = S // 2
    C = _CHUNKS_PER_B
    rows = S_half // C
    n_chunks = B * C

    O2 = O.reshape(B, S, K)

    def body(o_ref, w_ref, out_ref, send_buf, recv_buf, send_sems, recv_sems):
        my_x = lax.axis_index("x")
        my_y = lax.axis_index("y")
        my_z = lax.axis_index("z")
        peer = (1 - my_x, my_y, my_z)

        barrier = pltpu.get_barrier_semaphore()
        pl.semaphore_signal(
            barrier, inc=1, device_id=peer, device_id_type=pl.DeviceIdType.MESH
        )
        pl.semaphore_wait(barrier, 1)

        w = w_ref[...].astype(jnp.bfloat16)
        my_lo = my_x * S_half
        peer_lo = (1 - my_x) * S_half

        def chunk_rdma(b, c):
            i = b * C + c
            return pltpu.make_async_remote_copy(
                src_ref=send_buf.at[b, c * rows:(c + 1) * rows],
                dst_ref=recv_buf.at[b, c * rows:(c + 1) * rows],
                send_sem=send_sems.at[i],
                recv_sem=recv_sems.at[i],
                device_id=peer,
                device_id_type=pl.DeviceIdType.MESH,
            )

        for b in range(B):
            for c in range(C):
                a = o_ref[b, pl.ds(peer_lo + c * rows, rows)].astype(
                    jnp.bfloat16
                )
                send_buf[b, c * rows:(c + 1) * rows] = jnp.dot(
                    a, w, preferred_element_type=jnp.float32
                ).astype(jnp.bfloat16)
                chunk_rdma(b, c).start()

        for b in range(B):
            a = o_ref[b, pl.ds(my_lo, S_half)].astype(jnp.bfloat16)
            out_ref[b] = jnp.dot(a, w, preferred_element_type=jnp.float32)

        for b in range(B):
            for c in range(C):
                chunk_rdma(b, c).wait_recv()
                sl = pl.ds(c * rows, rows)
                out_ref[b, sl] = out_ref[b, sl] + recv_buf[
                    b, c * rows:(c + 1) * rows
                ].astype(jnp.float32)

        for b in range(B):
            for c in range(C):
                chunk_rdma(b, c).wait_send()

    return pl.pallas_call(
        body,
        out_shape=jax.ShapeDtypeStruct((B, S_half, N), jnp.float32),
        in_specs=[
            pl.BlockSpec(memory_space=pltpu.VMEM),
            pl.BlockSpec(memory_space=pltpu.VMEM),
        ],
        out_specs=pl.BlockSpec(memory_space=pltpu.VMEM),
        scratch_shapes=[
            pltpu.VMEM((B, S_half, N), jnp.bfloat16),
            pltpu.VMEM((B, S_half, N), jnp.bfloat16),
            pltpu.SemaphoreType.DMA((n_chunks,)),
            pltpu.SemaphoreType.DMA((n_chunks,)),
        ],
        compiler_params=pltpu.CompilerParams(collective_id=0),
    )(O2, Wo)


# device time: 59979 ns/iter; 1.1282x vs baseline; 1.0514x over previous
import jax
import jax.numpy as jnp
from jax import lax
from jax.experimental import pallas as pl
from jax.experimental.pallas import tpu as pltpu

_CHUNKS_PER_B = 4


def kernel(O, Wo):
    B, S, H, D = O.shape
    K = H * D
    N = Wo.shape[1]
    S_half = S // 2
    C = _CHUNKS_PER_B
    rows = S_half // C
    n_chunks = B * C

    Ot = jnp.transpose(O.reshape(B, S, K), (0, 2, 1))

    dn = (((0,), (0,)), ((), ()))

    def body(ot_ref, w_ref, out_ref, send_buf, recv_buf, acc_buf,
             send_sems, recv_sems, copy_sems):
        my_x = lax.axis_index("x")
        my_y = lax.axis_index("y")
        my_z = lax.axis_index("z")
        peer = (1 - my_x, my_y, my_z)

        barrier = pltpu.get_barrier_semaphore()
        pl.semaphore_signal(
            barrier, inc=1, device_id=peer, device_id_type=pl.DeviceIdType.MESH
        )
        pl.semaphore_wait(barrier, 1)

        w = w_ref[...].astype(jnp.bfloat16)
        my_lo = my_x * S_half
        peer_lo = (1 - my_x) * S_half

        def partial_chunk(b, lo, c):
            a_t = ot_ref[b, :, pl.ds(lo + c * rows, rows)].astype(jnp.bfloat16)
            return lax.dot_general(
                a_t, w, dn, preferred_element_type=jnp.float32
            )

        def chunk_rdma(b, c):
            i = b * C + c
            return pltpu.make_async_remote_copy(
                src_ref=send_buf.at[b, c * rows:(c + 1) * rows],
                dst_ref=recv_buf.at[b, c * rows:(c + 1) * rows],
                send_sem=send_sems.at[i],
                recv_sem=recv_sems.at[i],
                device_id=peer,
                device_id_type=pl.DeviceIdType.MESH,
            )

        for b in range(B):
            for c in range(C):
                send_buf[b, c * rows:(c + 1) * rows] = partial_chunk(
                    b, peer_lo, c
                ).astype(jnp.bfloat16)
                chunk_rdma(b, c).start()

        for b in range(B):
            for c in range(C):
                acc_buf[b, c * rows:(c + 1) * rows] = partial_chunk(b, my_lo, c)

        for b in range(B):
            for c in range(C):
                chunk_rdma(b, c).wait_recv()
                sl = pl.ds(c * rows, rows)
                acc_buf[b, sl] = acc_buf[b, sl] + recv_buf[
                    b, c * rows:(c + 1) * rows
                ].astype(jnp.float32)
                pltpu.make_async_copy(
                    acc_buf.at[b, c * rows:(c + 1) * rows],
                    out_ref.at[b, c * rows:(c + 1) * rows],
                    copy_sems.at[b * C + c],
                ).start()

        for b in range(B):
            for c in range(C):
                pltpu.make_async_copy(
                    acc_buf.at[b, c * rows:(c + 1) * rows],
                    out_ref.at[b, c * rows:(c + 1) * rows],
                    copy_sems.at[b * C + c],
                ).wait()
                chunk_rdma(b, c).wait_send()

    return pl.pallas_call(
        body,
        out_shape=jax.ShapeDtypeStruct((B, S_half, N), jnp.float32),
        in_specs=[
            pl.BlockSpec(memory_space=pltpu.VMEM),
            pl.BlockSpec(memory_space=pltpu.VMEM),
        ],
        out_specs=pl.BlockSpec(memory_space=pl.ANY),
        scratch_shapes=[
            pltpu.VMEM((B, S_half, N), jnp.bfloat16),
            pltpu.VMEM((B, S_half, N), jnp.bfloat16),
            pltpu.VMEM((B, S_half, N), jnp.float32),
            pltpu.SemaphoreType.DMA((n_chunks,)),
            pltpu.SemaphoreType.DMA((n_chunks,)),
            pltpu.SemaphoreType.DMA((n_chunks,)),
        ],
        compiler_params=pltpu.CompilerParams(collective_id=0),
    )(Ot, Wo)
